# baseline (speedup 1.0000x reference)
import jax
import jax.numpy as jnp
from jax.experimental import pallas as pl

SEG = 2048
K = 8
ROW_BLK = 256
ROW_SUB = 128
M_BLK = 1024
M_SUB = 256


def _topk_softmax(w):
    work = w
    mask = jnp.zeros(w.shape, dtype=jnp.bool_)
    row_max = None
    for i in range(K):
        m = jnp.max(work, axis=1, keepdims=True)
        if i == 0:
            row_max = jnp.maximum(m, 0.0)
        sel = work == m
        mask = jnp.logical_or(mask, sel)
        work = jnp.where(sel, -jnp.inf, work)
    masked = jnp.where(mask, w, 0.0)
    e = jnp.exp(masked - row_max)
    z = jnp.sum(e, axis=1, keepdims=True)
    return (e / z).astype(jnp.bfloat16)


def _weff_kernel(w_ref, op_ref, out_ref):
    op = op_ref[...]
    for j in range(ROW_BLK // ROW_SUB):
        sl = pl.ds(j * ROW_SUB, ROW_SUB)
        adj = _topk_softmax(w_ref[sl, :])
        out_ref[sl, :] = jnp.dot(
            adj, op, preferred_element_type=jnp.float32
        ).astype(jnp.bfloat16)


def _mm_kernel(x_ref, w_ref, o_ref):
    wb = w_ref[...]
    for j in range(M_BLK // M_SUB):
        sl = pl.ds(j * M_SUB, M_SUB)
        xb = x_ref[sl, :].astype(jnp.bfloat16)
        o_ref[sl, :] = jnp.dot(xb, wb, preferred_element_type=jnp.float32)


def kernel(x, adj_weight, out_proj):
    B, T, S = x.shape
    opb = out_proj.astype(jnp.bfloat16)
    w_eff = pl.pallas_call(
        _weff_kernel,
        grid=(SEG // ROW_BLK,),
        in_specs=[
            pl.BlockSpec((ROW_BLK, SEG), lambda i: (i, 0)),
            pl.BlockSpec((SEG, SEG), lambda i: (0, 0)),
        ],
        out_specs=pl.BlockSpec((ROW_BLK, SEG), lambda i: (i, 0)),
        out_shape=jax.ShapeDtypeStruct((SEG, SEG), jnp.bfloat16),
    )(adj_weight, opb)
    xm = x.reshape(B * T, S)
    out = pl.pallas_call(
        _mm_kernel,
        grid=(B * T // M_BLK,),
        in_specs=[
            pl.BlockSpec((M_BLK, S), lambda i: (i, 0)),
            pl.BlockSpec((S, S), lambda i: (0, 0)),
        ],
        out_specs=pl.BlockSpec((M_BLK, S), lambda i: (i, 0)),
        out_shape=jax.ShapeDtypeStruct((B * T, S), jnp.float32),
    )(xm, w_eff)
    return out.reshape(B, T, S)


# speedup vs baseline: 1.0677x; 1.0677x over previous
import jax
import jax.numpy as jnp
from jax.experimental import pallas as pl
from jax.experimental.pallas import tpu as pltpu

SEG = 2048
K = 8
ROW_SUB = 128
M_BLK = 256
M_SUB = 128


def _topk_softmax(w):
    work = w.astype(jnp.bfloat16)
    row_max = None
    for i in range(K):
        m = jnp.max(work, axis=1, keepdims=True)
        if i == 0:
            row_max = jnp.maximum(m.astype(jnp.float32), 0.0)
        work = jnp.where(work == m, -jnp.inf, work)
    keep = work == -jnp.inf
    masked = jnp.where(keep, w, 0.0)
    e = jnp.exp(masked - row_max)
    z = jnp.sum(e, axis=1, keepdims=True)
    return (e / z).astype(jnp.bfloat16)


def _fused_kernel(w_ref, opb_ref, x_ref, o_ref, weff_ref):
    @pl.when(pl.program_id(0) == 0)
    def _():
        opb = opb_ref[...]
        for b in range(SEG // ROW_SUB):
            sl = pl.ds(b * ROW_SUB, ROW_SUB)
            adj = _topk_softmax(w_ref[sl, :])
            weff_ref[sl, :] = jnp.dot(
                adj, opb, preferred_element_type=jnp.float32
            ).astype(jnp.bfloat16)

    wb = weff_ref[...]
    for j in range(M_BLK // M_SUB):
        sl = pl.ds(j * M_SUB, M_SUB)
        xb = x_ref[sl, :].astype(jnp.bfloat16)
        o_ref[sl, :] = jnp.dot(xb, wb, preferred_element_type=jnp.float32)


def kernel(x, adj_weight, out_proj):
    B, T, S = x.shape
    xm = x.reshape(B * T, S)
    opb = out_proj.astype(jnp.bfloat16)
    out = pl.pallas_call(
        _fused_kernel,
        grid=(B * T // M_BLK,),
        in_specs=[
            pl.BlockSpec((SEG, SEG), lambda i: (0, 0)),
            pl.BlockSpec((SEG, SEG), lambda i: (0, 0)),
            pl.BlockSpec((M_BLK, S), lambda i: (i, 0)),
        ],
        out_specs=pl.BlockSpec((M_BLK, S), lambda i: (i, 0)),
        out_shape=jax.ShapeDtypeStruct((B * T, S), jnp.float32),
        scratch_shapes=[pltpu.VMEM((SEG, SEG), jnp.bfloat16)],
    )(adj_weight, opb, xm)
    return out.reshape(B, T, S)


# speedup vs baseline: 1.1654x; 1.0915x over previous
import jax
import jax.numpy as jnp
from jax.experimental import pallas as pl
from jax.experimental.pallas import tpu as pltpu

SEG = 2048
K = 8
ROW_BLK = 1024
ROW_SUB = 128
M_BLK = 1024
M_SUB = 512


def _topk_softmax(w):
    work = w.astype(jnp.bfloat16)
    row_max = None
    for i in range(K):
        m = jnp.max(work, axis=1, keepdims=True)
        if i == 0:
            row_max = jnp.maximum(m.astype(jnp.float32), 0.0)
        work = jnp.where(work == m, -jnp.inf, work)
    keep = work == -jnp.inf
    masked = jnp.where(keep, w, 0.0)
    e = jnp.exp(masked - row_max)
    z = jnp.sum(e, axis=1, keepdims=True)
    return (e / z).astype(jnp.bfloat16)


def _weff_kernel(w_ref, op_ref, out_ref, opb_ref):
    @pl.when(pl.program_id(0) == 0)
    def _():
        opb_ref[...] = op_ref[...].astype(jnp.bfloat16)

    op = opb_ref[...]
    for j in range(ROW_BLK // ROW_SUB):
        sl = pl.ds(j * ROW_SUB, ROW_SUB)
        adj = _topk_softmax(w_ref[sl, :])
        out_ref[sl, :] = jnp.dot(
            adj, op, preferred_element_type=jnp.float32
        ).astype(jnp.bfloat16)


def _mm_kernel(x_ref, w_ref, o_ref):
    wb = w_ref[...]
    for j in range(M_BLK // M_SUB):
        sl = pl.ds(j * M_SUB, M_SUB)
        xb = x_ref[sl, :].astype(jnp.bfloat16)
        o_ref[sl, :] = jnp.dot(xb, wb, preferred_element_type=jnp.float32)


def kernel(x, adj_weight, out_proj):
    B, T, S = x.shape
    w_eff = pl.pallas_call(
        _weff_kernel,
        grid=(SEG // ROW_BLK,),
        in_specs=[
            pl.BlockSpec((ROW_BLK, SEG), lambda i: (i, 0)),
            pl.BlockSpec((SEG, SEG), lambda i: (0, 0)),
        ],
        out_specs=pl.BlockSpec((ROW_BLK, SEG), lambda i: (i, 0)),
        out_shape=jax.ShapeDtypeStruct((SEG, SEG), jnp.bfloat16),
        scratch_shapes=[pltpu.VMEM((SEG, SEG), jnp.bfloat16)],
    )(adj_weight, out_proj)
    xm = x.reshape(B * T, S)
    out = pl.pallas_call(
        _mm_kernel,
        grid=(B * T // M_BLK,),
        in_specs=[
            pl.BlockSpec((M_BLK, S), lambda i: (i, 0)),
            pl.BlockSpec((S, S), lambda i: (0, 0)),
        ],
        out_specs=pl.BlockSpec((M_BLK, S), lambda i: (i, 0)),
        out_shape=jax.ShapeDtypeStruct((B * T, S), jnp.float32),
    )(xm, w_eff)
    return out.reshape(B, T, S)
